# Initial kernel scaffold; baseline (speedup 1.0000x reference)
import jax
import jax.numpy as jnp
from jax.experimental import pallas as pl


def kernel(x, pos, batch, params):
    raise NotImplementedError("write your pallas kernel here")


# speedup vs baseline: 32.7684x; 32.7684x over previous
import functools
import math

import jax
import jax.numpy as jnp
import numpy as np
from jax.experimental import pallas as pl

_B = 8
_N = 2048
_K = 64
_S1, _S1P = 410, 416
_S2, _S2P = 103, 104
_N2P = 512
_TQ = 8

_HI = jax.lax.Precision.HIGHEST
_INTERPRET = False


def _dot(a, b):
    return jax.lax.dot_general(a, b, (((1,), (0,)), ((), ())),
                               precision=_HI, preferred_element_type=jnp.float32)


def _fps_body(npts, nvalid, sp, pos_ref, qpos_ref):
    px = pos_ref[0]
    py = pos_ref[1]
    pz = pos_ref[2]
    lane = jax.lax.broadcasted_iota(jnp.int32, (_B, npts), 1)
    lvalid = lane < nvalid
    col = jax.lax.broadcasted_iota(jnp.int32, (_B, sp), 1)

    def dist_to(lx, ly, lz):
        dx = px - lx
        dy = py - ly
        dz = pz - lz
        return (dx * dx + dy * dy) + dz * dz

    x0 = px[:, 0:1]
    y0 = py[:, 0:1]
    z0 = pz[:, 0:1]
    d0 = dist_to(x0, y0, z0)
    dists0 = jnp.where(lvalid, d0, -jnp.inf)
    qx0 = jnp.where(col == 0, x0, 0.0)
    qy0 = jnp.where(col == 0, y0, 0.0)
    qz0 = jnp.where(col == 0, z0, 0.0)

    def body(i, st):
        dists, qx, qy, qz = st
        m = jnp.max(dists, axis=1, keepdims=True)
        nxt = jnp.min(jnp.where(dists == m, lane, npts), axis=1, keepdims=True)
        oh = lane == nxt
        lx = jnp.sum(jnp.where(oh, px, 0.0), axis=1, keepdims=True)
        ly = jnp.sum(jnp.where(oh, py, 0.0), axis=1, keepdims=True)
        lz = jnp.sum(jnp.where(oh, pz, 0.0), axis=1, keepdims=True)
        hit = col == i
        qx = jnp.where(hit, lx, qx)
        qy = jnp.where(hit, ly, qy)
        qz = jnp.where(hit, lz, qz)
        d = jnp.where(lvalid, dist_to(lx, ly, lz), jnp.inf)
        return (jnp.minimum(dists, d), qx, qy, qz)

    _, qx, qy, qz = jax.lax.fori_loop(1, sp, body, (dists0, qx0, qy0, qz0))
    qpos_ref[0] = qx
    qpos_ref[1] = qy
    qpos_ref[2] = qz


def _fps_call(pos3, sp, nvalid):
    npts = pos3.shape[2]
    return pl.pallas_call(
        functools.partial(_fps_body, npts, nvalid, sp),
        grid=(1,),
        in_specs=[pl.BlockSpec((3, _B, npts), lambda i: (0, 0, 0))],
        out_specs=pl.BlockSpec((3, _B, sp), lambda i: (0, 0, 0)),
        out_shape=jax.ShapeDtypeStruct((3, _B, sp), jnp.float32),
        interpret=_INTERPRET,
    )(pos3)


def _sa_body(npts, nvalid, r2, cin, cout, pos_t_ref, qpos_r_ref, feat_ref,
             w1, b1, w2, b2, w3, b3, out_ref):
    cf = cin + 3
    pxyz = pos_t_ref[0]
    qr = qpos_r_ref[0]
    px = pxyz[0:1, :]
    py = pxyz[1:2, :]
    pz = pxyz[2:3, :]
    qx = qr[:, 0:1]
    qy = qr[:, 1:2]
    qz = qr[:, 2:3]
    dx = qx - px
    dy = qy - py
    dz = qz - pz
    d2 = (dx * dx + dy * dy) + dz * dz
    lane = jax.lax.broadcasted_iota(jnp.int32, (_TQ, npts), 1)
    mask = (d2 <= r2) & (lane < nvalid)

    c = jnp.where(mask, 1.0, 0.0)
    sh = 1
    while sh < npts:
        c = c + jnp.concatenate(
            [jnp.zeros((_TQ, sh), jnp.float32), c[:, :npts - sh]], axis=1)
        sh *= 2
    count = c[:, npts - 1:npts]
    cc = jnp.where(mask, jnp.minimum(c, float(_K + 2)), 0.0)

    kcol = jax.lax.broadcasted_iota(jnp.int32, (_K, 1), 0).astype(jnp.float32) + 1.0
    rows = []
    for q in range(_TQ):
        ccq = jax.lax.broadcast_in_dim(cc[q:q + 1, :], (_K, npts), (0, 1))
        rows.append(jnp.where(ccq == kcol, 1.0, 0.0))
    pall = jnp.concatenate(rows, axis=0)

    g = _dot(pall, feat_ref[0])

    rio = jax.lax.broadcasted_iota(jnp.int32, (_TQ * _K, 1), 0)
    ei = jnp.where(
        (rio >> 6) == jax.lax.broadcasted_iota(jnp.int32, (_TQ * _K, _TQ), 1),
        1.0, 0.0)
    qrow = _dot(ei, qr)
    sub = jnp.concatenate(
        [jnp.zeros((_TQ * _K, cin), jnp.float32), qrow], axis=1)
    h = g - sub
    h = jnp.maximum(_dot(h, w1[...]) + b1[...], 0.0)
    h = jnp.maximum(_dot(h, w2[...]) + b2[...], 0.0)
    h = jnp.maximum(_dot(h, w3[...]) + b3[...], 0.0)

    countrow = _dot(ei, count)
    krow = (rio & (_K - 1)).astype(jnp.float32)
    pen = jnp.where(krow < countrow, 0.0, -jnp.inf)
    h = h + pen
    out_ref[0] = jnp.max(h.reshape(_TQ, _K, cout), axis=1)


def _sa_call(pos_t, qpos_r, feat, ps, r, nvalid):
    npts = pos_t.shape[2]
    sp = qpos_r.shape[1]
    cin = feat.shape[2] - 3
    cout = ps[2][0].shape[1]
    r2 = np.float32(r * r)
    wb = []
    w_specs = []
    for w, b in ps:
        wb += [w, b.reshape(1, -1)]
        w_specs += [
            pl.BlockSpec(w.shape, lambda bb, t: (0, 0)),
            pl.BlockSpec((1, b.shape[0]), lambda bb, t: (0, 0)),
        ]
    return pl.pallas_call(
        functools.partial(_sa_body, npts, nvalid, r2, cin, cout),
        grid=(_B, sp // _TQ),
        in_specs=[
            pl.BlockSpec((1, 3, npts), lambda bb, t: (bb, 0, 0)),
            pl.BlockSpec((1, _TQ, 3), lambda bb, t: (bb, t, 0)),
            pl.BlockSpec((1, npts, cin + 3), lambda bb, t: (bb, 0, 0)),
        ] + w_specs,
        out_specs=pl.BlockSpec((1, _TQ, cout), lambda bb, t: (bb, t, 0)),
        out_shape=jax.ShapeDtypeStruct((_B, sp, cout), jnp.float32),
        interpret=_INTERPRET,
    )(pos_t, qpos_r, feat, *wb)


def _gsa_body(x2_ref, q2_ref, w1, b1, w2, b2, w3, b3, out_ref):
    rows = _B * _S2P
    x2 = x2_ref[...].reshape(rows, x2_ref.shape[2])
    q2 = q2_ref[...].reshape(rows, 3)
    h = jnp.concatenate([x2, q2], axis=1)
    h = jnp.maximum(_dot(h, w1[...]) + b1[...], 0.0)
    h = jnp.maximum(_dot(h, w2[...]) + b2[...], 0.0)
    h = jnp.maximum(_dot(h, w3[...]) + b3[...], 0.0)
    cout = h.shape[1]
    h3 = h.reshape(_B, _S2P, cout)
    sio3 = jax.lax.broadcasted_iota(jnp.int32, (_B, _S2P, cout), 1)
    hm = jnp.where(sio3 < _S2, h3, -jnp.inf)
    out_ref[...] = jnp.max(hm, axis=1)


def _gsa_call(x2, qpos2_r, ps):
    cout = ps[2][0].shape[1]
    wb = []
    w_specs = []
    for w, b in ps:
        wb += [w, b.reshape(1, -1)]
        w_specs += [
            pl.BlockSpec(w.shape, lambda i: (0, 0)),
            pl.BlockSpec((1, b.shape[0]), lambda i: (0, 0)),
        ]
    return pl.pallas_call(
        _gsa_body,
        grid=(1,),
        in_specs=[
            pl.BlockSpec(x2.shape, lambda i: (0, 0, 0)),
            pl.BlockSpec(qpos2_r.shape, lambda i: (0, 0, 0)),
        ] + w_specs,
        out_specs=pl.BlockSpec((_B, cout), lambda i: (0, 0)),
        out_shape=jax.ShapeDtypeStruct((_B, cout), jnp.float32),
        interpret=_INTERPRET,
    )(x2, qpos2_r, *wb)


def kernel(x, pos, batch, params):
    xb = x.reshape(_B, _N, -1)
    posb = pos.reshape(_B, _N, 3)
    pos_t1 = jnp.transpose(posb, (0, 2, 1))
    fps_in1 = jnp.transpose(posb, (2, 0, 1))

    qpos1_t = _fps_call(fps_in1, _S1P, _N)
    qpos1_r = jnp.transpose(qpos1_t, (1, 2, 0))
    feat1 = jnp.concatenate([xb, posb], axis=-1)
    x1 = _sa_call(pos_t1, qpos1_r, feat1, params['sa1'], 0.2, _N)

    q2in = jnp.pad(qpos1_t[:, :, :_S1],
                   ((0, 0), (0, 0), (0, _N2P - _S1)))
    qpos2_t = _fps_call(q2in, _S2P, _S1)
    qpos2_r = jnp.transpose(qpos2_t, (1, 2, 0))
    pos_t2 = jnp.transpose(q2in, (1, 0, 2))
    feat2 = jnp.pad(
        jnp.concatenate([x1[:, :_S1], qpos1_r[:, :_S1]], axis=-1),
        ((0, 0), (0, _N2P - _S1), (0, 0)))
    x2 = _sa_call(pos_t2, qpos2_r, feat2, params['sa2'], 0.4, _S1)

    return _gsa_call(x2, qpos2_r, params['gsa'])
